# baseline (speedup 1.0000x reference)
import functools

import jax
import jax.numpy as jnp
from jax import lax
from jax.experimental import pallas as pl
from jax.experimental.pallas import tpu as pltpu
from jax.experimental.pallas import tpu_sc as plsc

_C = 128
_L = 16
_NT = 32
_NS = 16


@functools.lru_cache(maxsize=None)
def _sc_message_kernel(Np, R):
    base_cnt, extra = R // _NT, R % _NT
    pairs, resid = base_cnt // 2, base_cnt % 2
    E = R * _C

    mesh = plsc.VectorSubcoreMesh(core_axis_name="c", subcore_axis_name="s")

    def body(ei_h, sh0_h, sh1_h, nodes_h, zeros_h, aw_h, out_h,
             tab0, tab1, tab2, tab3, acc0, acc1, acc2, acc3,
             src_v0, tgt_v0, sh0_v0, sh1_v0,
             f0_0, f1_0, f2_0, f3_0, m0_0, m1_0, m2_0, m3_0,
             src_v1, tgt_v1, sh0_v1, sh1_v1,
             f0_1, f1_1, f2_1, f3_1, m0_1, m1_1, m2_1, m3_1,
             aw_v,
             lsem0, lsem1, gsem0, gsem1, ssem0, ssem1):
        c = lax.axis_index("c")
        s = lax.axis_index("s")
        wid = c * _NS + s
        tabs = [tab0, tab1, tab2, tab3]
        accs = [acc0, acc1, acc2, acc3]
        slots = [
            dict(src=src_v0, tgt=tgt_v0, sh0=sh0_v0, sh1=sh1_v0,
                 fs=[f0_0, f1_0, f2_0, f3_0], ms=[m0_0, m1_0, m2_0, m3_0],
                 lsem=lsem0, gsem=gsem0, ssem=ssem0),
            dict(src=src_v1, tgt=tgt_v1, sh0=sh0_v1, sh1=sh1_v1,
                 fs=[f0_1, f1_1, f2_1, f3_1], ms=[m0_1, m1_1, m2_1, m3_1],
                 lsem=lsem1, gsem=gsem1, ssem=ssem1),
        ]

        @pl.when(s == 0)
        def _():
            for k in range(4):
                pltpu.sync_copy(nodes_h.at[pl.ds(k * Np, Np)], tabs[k])
                pltpu.sync_copy(zeros_h, accs[k])
        pltpu.sync_copy(aw_h, aw_v)
        plsc.subcore_barrier()

        A0 = aw_v[0]
        A1 = aw_v[1]
        A2 = aw_v[2]
        A3 = aw_v[3]
        A4 = aw_v[4]

        def load_copies(b, r):
            e0 = pl.multiple_of(r * _C, _C)
            e1 = pl.multiple_of(r * (3 * _C), 3 * _C)
            return [
                pltpu.make_async_copy(ei_h.at[pl.ds(e0, _C)], b["src"], b["lsem"]),
                pltpu.make_async_copy(ei_h.at[pl.ds(e0 + E, _C)], b["tgt"], b["lsem"]),
                pltpu.make_async_copy(sh0_h.at[pl.ds(e0, _C)], b["sh0"], b["lsem"]),
                pltpu.make_async_copy(sh1_h.at[pl.ds(e1, 3 * _C)], b["sh1"], b["lsem"]),
            ]

        def gather_copies(b):
            return [pltpu.make_async_copy(tabs[k].at[b["src"]], b["fs"][k], b["gsem"])
                    for k in range(4)]

        def scatter_copies(b):
            return [pltpu.make_async_copy(b["ms"][k], accs[k].at[b["tgt"]], b["ssem"])
                    for k in range(4)]

        def issue_loads(b, r):
            for d in load_copies(b, r):
                d.start()

        def wait_loads(b, r):
            for d in load_copies(b, r):
                d.wait()

        def compute(b):
            for g in range(_C // _L):
                sl = pl.ds(g * _L, _L)
                sv = b["fs"][0][sl]
                v1 = b["fs"][1][sl]
                v2 = b["fs"][2][sl]
                v3 = b["fs"][3][sl]
                u1 = b["sh1"][pl.ds(g * _L, _L)]
                u2 = b["sh1"][pl.ds(_C + g * _L, _L)]
                u3 = b["sh1"][pl.ds(2 * _C + g * _L, _L)]
                t0 = b["sh0"][sl]
                uv = u1 * v1 + u2 * v2 + u3 * v3
                b["ms"][0][sl] = A0 * (t0 * sv) + A3 * uv
                b["ms"][1][sl] = A1 * (t0 * v1) + A2 * (sv * u1) + A4 * (u2 * v3 - u3 * v2)
                b["ms"][2][sl] = A1 * (t0 * v2) + A2 * (sv * u2) + A4 * (u3 * v1 - u1 * v3)
                b["ms"][3][sl] = A1 * (t0 * v3) + A2 * (sv * u3) + A4 * (u1 * v2 - u2 * v1)

        def chunk_sync(r):
            b = slots[0]
            wait_loads(b, r)
            for d in gather_copies(b):
                d.start()
            for d in gather_copies(b):
                d.wait()
            compute(b)
            for d in scatter_copies(b):
                d.start(add=True)
            for d in scatter_copies(b):
                d.wait()

        base = wid * base_cnt
        if pairs:
            issue_loads(slots[0], base)

            def pair(i, carry):
                r0 = base + 2 * i
                b0, b1 = slots
                wait_loads(b0, r0)
                for d in gather_copies(b0):
                    d.start()
                @pl.when(i > 0)
                def _():
                    for d in scatter_copies(b1):
                        d.wait()
                issue_loads(b1, r0 + 1)
                for d in gather_copies(b0):
                    d.wait()
                compute(b0)
                for d in scatter_copies(b0):
                    d.start(add=True)
                wait_loads(b1, r0 + 1)
                for d in gather_copies(b1):
                    d.start()
                @pl.when(i < pairs - 1)
                def _():
                    for d in scatter_copies(b0):
                        d.wait()
                    issue_loads(b0, r0 + 2)
                for d in gather_copies(b1):
                    d.wait()
                compute(b1)
                for d in scatter_copies(b1):
                    d.start(add=True)
                return carry

            lax.fori_loop(0, pairs, pair, 0)
            for d in scatter_copies(slots[0]):
                d.wait()
            for d in scatter_copies(slots[1]):
                d.wait()
        if resid:
            issue_loads(slots[0], base + base_cnt - 1)
            chunk_sync(base + base_cnt - 1)
        if extra:
            @pl.when(wid < extra)
            def _():
                issue_loads(slots[0], _NT * base_cnt + wid)
                chunk_sync(_NT * base_cnt + wid)

        plsc.subcore_barrier()
        @pl.when(s == 0)
        def _():
            for k in range(4):
                off = pl.multiple_of((c * 4 + k) * Np, 128)
                pltpu.sync_copy(accs[k], out_h.at[pl.ds(off, Np)])

    slot_scratch = [
        pltpu.VMEM((_C,), jnp.int32),
        pltpu.VMEM((_C,), jnp.int32),
        pltpu.VMEM((_C,), jnp.float32),
        pltpu.VMEM((3 * _C,), jnp.float32),
        pltpu.VMEM((_C,), jnp.float32),
        pltpu.VMEM((_C,), jnp.float32),
        pltpu.VMEM((_C,), jnp.float32),
        pltpu.VMEM((_C,), jnp.float32),
        pltpu.VMEM((_C,), jnp.float32),
        pltpu.VMEM((_C,), jnp.float32),
        pltpu.VMEM((_C,), jnp.float32),
        pltpu.VMEM((_C,), jnp.float32),
    ]

    return pl.kernel(
        body,
        out_type=jax.ShapeDtypeStruct((8 * Np,), jnp.float32),
        mesh=mesh,
        scratch_types=(
            [pltpu.VMEM_SHARED((Np,), jnp.float32)] * 4
            + [pltpu.VMEM_SHARED((Np,), jnp.float32)] * 4
            + slot_scratch
            + slot_scratch
            + [pltpu.VMEM((5, _L), jnp.float32)]
            + [pltpu.SemaphoreType.DMA] * 6
        ),
    )


def _perm_matrix():
    import numpy as np
    P = np.zeros((3 * _C, 3 * _C), np.float32)
    for t in range(_C):
        for k in range(3):
            P[3 * t + k, k * _C + t] = 1.0
    return jnp.asarray(P)


def _tc_perm(x):
    R = x.shape[0]
    B = 2000
    while B > 8 and R % B:
        B -= 8
    assert R % B == 0 and B % 8 == 0
    P = _perm_matrix()

    def body(x_ref, p_ref, o_ref):
        o_ref[...] = jax.lax.dot(
            x_ref[...], p_ref[...],
            preferred_element_type=jnp.float32,
            precision=jax.lax.Precision.HIGHEST)

    return pl.pallas_call(
        body,
        grid=(R // B,),
        in_specs=[
            pl.BlockSpec((B, 3 * _C), lambda i: (i, 0)),
            pl.BlockSpec((3 * _C, 3 * _C), lambda i: (0, 0)),
        ],
        out_specs=pl.BlockSpec((B, 3 * _C), lambda i: (i, 0)),
        out_shape=jax.ShapeDtypeStruct((R, 3 * _C), jnp.float32),
    )(x, P)


def _tc_add(p):
    def body(p_ref, o_ref):
        o_ref[...] = p_ref[0] + p_ref[1]
    return pl.pallas_call(
        body,
        out_shape=jax.ShapeDtypeStruct(p.shape[1:], p.dtype),
    )(p)


def kernel(node_irreps, sh_edge_0, sh_edge_1, weights, edge_index):
    N = node_irreps.shape[0]
    E = edge_index.shape[1]
    assert E % _C == 0
    R = E // _C
    Np = -(-N // 128) * 128
    ei = edge_index.reshape(2 * E)
    sh0 = sh_edge_0.reshape(E)
    sh1 = _tc_perm(sh_edge_1.astype(jnp.float32).reshape(R, 3 * _C)).reshape(3 * E)
    nodes = jnp.zeros((4, Np), jnp.float32).at[:, :N].set(
        node_irreps.astype(jnp.float32).T).reshape(4 * Np)
    scale = jnp.array([1.0, 3.0 ** -0.5, 3.0 ** -0.5, 3.0 ** -0.5, 6.0 ** -0.5],
                      jnp.float32)
    aw = jnp.broadcast_to((weights * scale)[:, None], (5, _L)).astype(jnp.float32)
    zeros = jnp.zeros((Np,), jnp.float32)
    partial = _sc_message_kernel(Np, R)(ei, sh0, sh1, nodes, zeros, aw)
    flat = partial.reshape(2, (4 * Np) // 128, 128)
    summed = _tc_add(flat).reshape(4, Np)
    return summed[:, :N].T


# speedup vs baseline: 3.9077x; 3.9077x over previous
import functools

import jax
import jax.numpy as jnp
from jax import lax
from jax.experimental import pallas as pl
from jax.experimental.pallas import tpu as pltpu
from jax.experimental.pallas import tpu_sc as plsc

_C = 128
_L = 16
_NT = 32
_NS = 16


@functools.lru_cache(maxsize=None)
def _sc_message_kernel(Np, R):
    base_cnt, extra = R // _NT, R % _NT
    pairs, resid = base_cnt // 2, base_cnt % 2
    E = R * _C

    mesh = plsc.VectorSubcoreMesh(core_axis_name="c", subcore_axis_name="s")

    def body(ei_h, sh0_h, sh1_h, nodes_h, zeros_h, aw_h, out_h,
             tab0, tab1, tab2, tab3, acc0, acc1, acc2, acc3,
             src_v0, tgt_v0, sh0_v0, u1_v0, u2_v0, u3_v0,
             f0_0, f1_0, f2_0, f3_0, m0_0, m1_0, m2_0, m3_0,
             src_v1, tgt_v1, sh0_v1, u1_v1, u2_v1, u3_v1,
             f0_1, f1_1, f2_1, f3_1, m0_1, m1_1, m2_1, m3_1,
             aw_v,
             lsem0, lsem1, gsem0, gsem1, ssem0, ssem1):
        c = lax.axis_index("c")
        s = lax.axis_index("s")
        wid = c * _NS + s
        tabs = [tab0, tab1, tab2, tab3]
        accs = [acc0, acc1, acc2, acc3]
        slots = [
            dict(src=src_v0, tgt=tgt_v0, sh0=sh0_v0, us=[u1_v0, u2_v0, u3_v0],
                 fs=[f0_0, f1_0, f2_0, f3_0], ms=[m0_0, m1_0, m2_0, m3_0],
                 lsem=lsem0, gsem=gsem0, ssem=ssem0),
            dict(src=src_v1, tgt=tgt_v1, sh0=sh0_v1, us=[u1_v1, u2_v1, u3_v1],
                 fs=[f0_1, f1_1, f2_1, f3_1], ms=[m0_1, m1_1, m2_1, m3_1],
                 lsem=lsem1, gsem=gsem1, ssem=ssem1),
        ]

        @pl.when(s == 0)
        def _():
            for k in range(4):
                pltpu.sync_copy(nodes_h.at[pl.ds(k * Np, Np)], tabs[k])
                pltpu.sync_copy(zeros_h, accs[k])
        pltpu.sync_copy(aw_h, aw_v)
        plsc.subcore_barrier()

        A0 = aw_v[0]
        A1 = aw_v[1]
        A2 = aw_v[2]
        A3 = aw_v[3]
        A4 = aw_v[4]

        def load_copies(b, r):
            e0 = pl.multiple_of(r * _C, _C)
            return [
                pltpu.make_async_copy(ei_h.at[pl.ds(e0, _C)], b["src"], b["lsem"]),
                pltpu.make_async_copy(ei_h.at[pl.ds(e0 + E, _C)], b["tgt"], b["lsem"]),
                pltpu.make_async_copy(sh0_h.at[pl.ds(e0, _C)], b["sh0"], b["lsem"]),
                pltpu.make_async_copy(sh1_h.at[pl.ds(e0, _C)], b["us"][0], b["lsem"]),
                pltpu.make_async_copy(sh1_h.at[pl.ds(e0 + E, _C)], b["us"][1], b["lsem"]),
                pltpu.make_async_copy(sh1_h.at[pl.ds(e0 + 2 * E, _C)], b["us"][2], b["lsem"]),
            ]

        def gather_copies(b):
            return [pltpu.make_async_copy(tabs[k].at[b["src"]], b["fs"][k], b["gsem"])
                    for k in range(4)]

        def scatter_copies(b):
            return [pltpu.make_async_copy(b["ms"][k], accs[k].at[b["tgt"]], b["ssem"])
                    for k in range(4)]

        def issue_loads(b, r):
            for d in load_copies(b, r):
                d.start()

        def wait_loads(b, r):
            for d in load_copies(b, r):
                d.wait()

        def compute(b):
            for g in range(_C // _L):
                sl = pl.ds(g * _L, _L)
                sv = b["fs"][0][sl]
                v1 = b["fs"][1][sl]
                v2 = b["fs"][2][sl]
                v3 = b["fs"][3][sl]
                u1 = b["us"][0][sl]
                u2 = b["us"][1][sl]
                u3 = b["us"][2][sl]
                t0 = b["sh0"][sl]
                uv = u1 * v1 + u2 * v2 + u3 * v3
                b["ms"][0][sl] = A0 * (t0 * sv) + A3 * uv
                b["ms"][1][sl] = A1 * (t0 * v1) + A2 * (sv * u1) + A4 * (u2 * v3 - u3 * v2)
                b["ms"][2][sl] = A1 * (t0 * v2) + A2 * (sv * u2) + A4 * (u3 * v1 - u1 * v3)
                b["ms"][3][sl] = A1 * (t0 * v3) + A2 * (sv * u3) + A4 * (u1 * v2 - u2 * v1)

        def chunk_sync(r):
            b = slots[0]
            wait_loads(b, r)
            for d in gather_copies(b):
                d.start()
            for d in gather_copies(b):
                d.wait()
            compute(b)
            for d in scatter_copies(b):
                d.start(add=True)
            for d in scatter_copies(b):
                d.wait()

        base = wid * base_cnt
        if pairs:
            issue_loads(slots[0], base)

            def pair(i, carry):
                r0 = base + 2 * i
                b0, b1 = slots
                wait_loads(b0, r0)
                for d in gather_copies(b0):
                    d.start()
                @pl.when(i > 0)
                def _():
                    for d in scatter_copies(b1):
                        d.wait()
                issue_loads(b1, r0 + 1)
                for d in gather_copies(b0):
                    d.wait()
                compute(b0)
                for d in scatter_copies(b0):
                    d.start(add=True)
                wait_loads(b1, r0 + 1)
                for d in gather_copies(b1):
                    d.start()
                @pl.when(i < pairs - 1)
                def _():
                    for d in scatter_copies(b0):
                        d.wait()
                    issue_loads(b0, r0 + 2)
                for d in gather_copies(b1):
                    d.wait()
                compute(b1)
                for d in scatter_copies(b1):
                    d.start(add=True)
                return carry

            lax.fori_loop(0, pairs, pair, 0)
            for d in scatter_copies(slots[0]):
                d.wait()
            for d in scatter_copies(slots[1]):
                d.wait()
        if resid:
            issue_loads(slots[0], base + base_cnt - 1)
            chunk_sync(base + base_cnt - 1)
        if extra:
            @pl.when(wid < extra)
            def _():
                issue_loads(slots[0], _NT * base_cnt + wid)
                chunk_sync(_NT * base_cnt + wid)

        plsc.subcore_barrier()
        @pl.when(s == 0)
        def _():
            for k in range(4):
                off = pl.multiple_of((c * 4 + k) * Np, 128)
                pltpu.sync_copy(accs[k], out_h.at[pl.ds(off, Np)])

    slot_scratch = [
        pltpu.VMEM((_C,), jnp.int32),
        pltpu.VMEM((_C,), jnp.int32),
        pltpu.VMEM((_C,), jnp.float32),
        pltpu.VMEM((_C,), jnp.float32),
        pltpu.VMEM((_C,), jnp.float32),
        pltpu.VMEM((_C,), jnp.float32),
        pltpu.VMEM((_C,), jnp.float32),
        pltpu.VMEM((_C,), jnp.float32),
        pltpu.VMEM((_C,), jnp.float32),
        pltpu.VMEM((_C,), jnp.float32),
        pltpu.VMEM((_C,), jnp.float32),
        pltpu.VMEM((_C,), jnp.float32),
        pltpu.VMEM((_C,), jnp.float32),
        pltpu.VMEM((_C,), jnp.float32),
    ]

    return pl.kernel(
        body,
        out_type=jax.ShapeDtypeStruct((8 * Np,), jnp.float32),
        mesh=mesh,
        scratch_types=(
            [pltpu.VMEM_SHARED((Np,), jnp.float32)] * 4
            + [pltpu.VMEM_SHARED((Np,), jnp.float32)] * 4
            + slot_scratch
            + slot_scratch
            + [pltpu.VMEM((5, _L), jnp.float32)]
            + [pltpu.SemaphoreType.DMA] * 6
        ),
    )


def _tc_add(p):
    def body(p_ref, o_ref):
        o_ref[...] = p_ref[0] + p_ref[1]
    return pl.pallas_call(
        body,
        out_shape=jax.ShapeDtypeStruct(p.shape[1:], p.dtype),
    )(p)


def kernel(node_irreps, sh_edge_0, sh_edge_1, weights, edge_index):
    N = node_irreps.shape[0]
    E = edge_index.shape[1]
    assert E % _C == 0
    R = E // _C
    Np = -(-N // 128) * 128
    ei = edge_index.reshape(2 * E)
    sh0 = sh_edge_0.reshape(E)
    sh1 = sh_edge_1.astype(jnp.float32).T.reshape(3 * E)
    nodes = jnp.zeros((4, Np), jnp.float32).at[:, :N].set(
        node_irreps.astype(jnp.float32).T).reshape(4 * Np)
    scale = jnp.array([1.0, 3.0 ** -0.5, 3.0 ** -0.5, 3.0 ** -0.5, 6.0 ** -0.5],
                      jnp.float32)
    aw = jnp.broadcast_to((weights * scale)[:, None], (5, _L)).astype(jnp.float32)
    zeros = jnp.zeros((Np,), jnp.float32)
    partial = _sc_message_kernel(Np, R)(ei, sh0, sh1, nodes, zeros, aw)
    flat = partial.reshape(2, (4 * Np) // 128, 128)
    summed = _tc_add(flat).reshape(4, Np)
    return summed[:, :N].T
